# Initial kernel scaffold; baseline (speedup 1.0000x reference)
import jax
import jax.numpy as jnp
from jax.experimental import pallas as pl


def kernel(x, conn, weights, bias):
    raise NotImplementedError("write your pallas kernel here")


# speedup vs baseline: 10.5245x; 10.5245x over previous
import functools

import jax
import jax.numpy as jnp
from jax import lax
from jax.experimental import pallas as pl
from jax.experimental.pallas import tpu as pltpu
from jax.experimental.pallas import tpu_sc as plsc

B, CIN, H, W = 4, 384, 24, 24
COUT, K = 384, 32
HW = H * W
HWP = 640
L = 16
NC, NS = 2, 16
NW = NC * NS
OPW = COUT // NW
NJ = HW // L
T = OPW * B
KH = K // 2


def _sc_body(xf_hbm, idx_hbm, wb_hbm, bb_hbm, out_hbm,
             idx_v, w_v, b_v, rows0, rows1, out_v, sem0, sem1):
    wid = lax.axis_index("s") * NC + lax.axis_index("c")

    pltpu.sync_copy(idx_hbm.at[wid], idx_v)
    pltpu.sync_copy(wb_hbm.at[wid], w_v)
    pltpu.sync_copy(bb_hbm.at[wid], b_v)

    def issue(tt, rows_ref, sem):
        oi = tt // B
        b = lax.rem(tt, B)
        pltpu.async_copy(xf_hbm.at[idx_v.at[b, oi]], rows_ref, sem)

    def wait_rows(rows_ref, sem):
        pltpu.make_async_copy(xf_hbm.at[pl.ds(0, K)], rows_ref, sem).wait()

    def compute(tt, rows_ref):
        oi = tt // B
        b = lax.rem(tt, B)
        row = b * OPW + oi
        bv = b_v[oi]

        w_lo = [w_v[oi, pl.ds(k * L, L)] for k in range(KH)]
        w_hi = [w_v[oi, pl.ds((KH + k) * L, L)] for k in range(KH)]

        def pass_lo(j, _):
            s = pl.ds(j * L, L)
            acc = jnp.abs(rows_ref[0, s] - w_lo[0])
            for k in range(1, KH):
                acc = jnp.maximum(acc, jnp.abs(rows_ref[k, s] - w_lo[k]))
            out_v[row, s] = acc
            return 0

        def pass_hi(j, _):
            s = pl.ds(j * L, L)
            acc = out_v[row, s]
            for k in range(KH):
                acc = jnp.maximum(acc, jnp.abs(rows_ref[KH + k, s] - w_hi[k]))
            out_v[row, s] = acc + bv
            return 0

        lax.fori_loop(0, NJ, pass_lo, 0)
        lax.fori_loop(0, NJ, pass_hi, 0)

    issue(0, rows0, sem0)

    def tbody(i, _):
        t0 = i * 2

        @pl.when(t0 + 1 < T)
        def _():
            issue(t0 + 1, rows1, sem1)

        wait_rows(rows0, sem0)
        compute(t0, rows0)

        @pl.when(t0 + 2 < T)
        def _():
            issue(t0 + 2, rows0, sem0)

        wait_rows(rows1, sem1)
        compute(t0 + 1, rows1)
        return 0

    lax.fori_loop(0, T // 2, tbody, 0)

    pltpu.sync_copy(out_v, out_hbm.at[wid])


@jax.jit
def _dist_conv(xf, idx_all, w_b, bias_b):
    mesh = plsc.VectorSubcoreMesh(core_axis_name="c", subcore_axis_name="s")
    call = functools.partial(
        pl.kernel,
        out_type=jax.ShapeDtypeStruct((NW, B * OPW, HW), jnp.float32),
        mesh=mesh,
        scratch_types=[
            pltpu.VMEM((B, OPW, K), jnp.int32),
            pltpu.VMEM((OPW, K * L), jnp.float32),
            pltpu.VMEM((OPW, L), jnp.float32),
            pltpu.VMEM((K, HWP), jnp.float32),
            pltpu.VMEM((K, HWP), jnp.float32),
            pltpu.VMEM((B * OPW, HW), jnp.float32),
            pltpu.SemaphoreType.DMA,
            pltpu.SemaphoreType.DMA,
        ],
    )(_sc_body)
    return call(xf, idx_all, w_b, bias_b)


def kernel(x, conn, weights, bias):
    xf = jnp.pad(x.reshape(B * CIN, HW), ((0, 0), (0, HWP - HW)))
    idx_all = (conn.reshape(COUT, K)[None, :, :]
               + (jnp.arange(B, dtype=jnp.int32) * CIN)[:, None, None])
    idx_all = idx_all.reshape(B, NW, OPW, K).transpose(1, 0, 2, 3)
    w_b = jnp.broadcast_to(weights[:, :, None], (COUT, K, L)).reshape(NW, OPW, K * L)
    bias_b = jnp.broadcast_to(bias.reshape(COUT, 1), (COUT, L)).reshape(NW, OPW, L)
    out = _dist_conv(xf, idx_all, w_b, bias_b)
    out = out.reshape(NW, B, OPW, HW).transpose(1, 0, 2, 3)
    return out.reshape(B, COUT, H, W)


# speedup vs baseline: 10.6678x; 1.0136x over previous
import functools

import jax
import jax.numpy as jnp
from jax import lax
from jax.experimental import pallas as pl
from jax.experimental.pallas import tpu as pltpu
from jax.experimental.pallas import tpu_sc as plsc

B, CIN, H, W = 4, 384, 24, 24
COUT, K = 384, 32
HW = H * W
HWP = 640
L = 16
NC, NS = 2, 16
NW = NC * NS
OPW = COUT // NW
NJ = HW // L
T = OPW * B
KH = K // 2


def _sc_body(xf_hbm, idx_hbm, wb_hbm, bb_hbm, out_hbm,
             idx_v, w_v, b_v, rows0, rows1, out_v, sem0, sem1):
    wid = lax.axis_index("s") * NC + lax.axis_index("c")

    pltpu.sync_copy(idx_hbm.at[wid], idx_v)
    pltpu.sync_copy(wb_hbm.at[wid], w_v)
    pltpu.sync_copy(bb_hbm.at[wid], b_v)

    def issue(tt, rows_ref, sem):
        oi = tt // B
        b = lax.rem(tt, B)
        pltpu.async_copy(xf_hbm.at[idx_v.at[b, oi]], rows_ref, sem)

    def wait_rows(rows_ref, sem):
        pltpu.make_async_copy(xf_hbm.at[pl.ds(0, K)], rows_ref, sem).wait()

    def compute(tt, rows_ref):
        oi = tt // B
        b = lax.rem(tt, B)
        row = b * OPW + oi
        bv = b_v[oi]

        ws = [w_v[oi, pl.ds(k * L, L)] for k in range(K)]

        def body(j, _):
            s = pl.ds(j * L, L)
            acc = None
            for g in range(0, K, 8):
                d = [jnp.abs(rows_ref[g + k, s] - ws[g + k]) for k in range(8)]
                t0 = jnp.maximum(jnp.maximum(d[0], d[1]), jnp.maximum(d[2], d[3]))
                t1 = jnp.maximum(jnp.maximum(d[4], d[5]), jnp.maximum(d[6], d[7]))
                t = jnp.maximum(t0, t1)
                acc = t if acc is None else jnp.maximum(acc, t)
            out_v[row, s] = acc + bv
            return 0

        lax.fori_loop(0, NJ, body, 0)

    issue(0, rows0, sem0)

    def tbody(i, _):
        t0 = i * 2

        @pl.when(t0 + 1 < T)
        def _():
            issue(t0 + 1, rows1, sem1)

        wait_rows(rows0, sem0)
        compute(t0, rows0)

        @pl.when(t0 + 2 < T)
        def _():
            issue(t0 + 2, rows0, sem0)

        wait_rows(rows1, sem1)
        compute(t0 + 1, rows1)
        return 0

    lax.fori_loop(0, T // 2, tbody, 0)

    pltpu.sync_copy(out_v, out_hbm.at[wid])


@jax.jit
def _dist_conv(xf, idx_all, w_b, bias_b):
    mesh = plsc.VectorSubcoreMesh(core_axis_name="c", subcore_axis_name="s")
    call = functools.partial(
        pl.kernel,
        out_type=jax.ShapeDtypeStruct((NW, B * OPW, HW), jnp.float32),
        mesh=mesh,
        scratch_types=[
            pltpu.VMEM((B, OPW, K), jnp.int32),
            pltpu.VMEM((OPW, K * L), jnp.float32),
            pltpu.VMEM((OPW, L), jnp.float32),
            pltpu.VMEM((K, HWP), jnp.float32),
            pltpu.VMEM((K, HWP), jnp.float32),
            pltpu.VMEM((B * OPW, HW), jnp.float32),
            pltpu.SemaphoreType.DMA,
            pltpu.SemaphoreType.DMA,
        ],
    )(_sc_body)
    return call(xf, idx_all, w_b, bias_b)


def kernel(x, conn, weights, bias):
    xf = jnp.pad(x.reshape(B * CIN, HW), ((0, 0), (0, HWP - HW)))
    idx_all = (conn.reshape(COUT, K)[None, :, :]
               + (jnp.arange(B, dtype=jnp.int32) * CIN)[:, None, None])
    idx_all = idx_all.reshape(B, NW, OPW, K).transpose(1, 0, 2, 3)
    w_b = jnp.broadcast_to(weights[:, :, None], (COUT, K, L)).reshape(NW, OPW, K * L)
    bias_b = jnp.broadcast_to(bias.reshape(COUT, 1), (COUT, L)).reshape(NW, OPW, L)
    out = _dist_conv(xf, idx_all, w_b, bias_b)
    out = out.reshape(NW, B, OPW, HW).transpose(1, 0, 2, 3)
    return out.reshape(B, COUT, H, W)


# speedup vs baseline: 11.5777x; 1.0853x over previous
import functools

import jax
import jax.numpy as jnp
from jax import lax
from jax.experimental import pallas as pl
from jax.experimental.pallas import tpu as pltpu
from jax.experimental.pallas import tpu_sc as plsc

B, CIN, H, W = 4, 384, 24, 24
COUT, K = 384, 32
HW = H * W
HWP = 640
L = 16
NC, NS = 2, 16
NW = NC * NS
OPW = COUT // NW
NJ = HW // L
T = OPW * B


def _sc_body(xf_hbm, conn_hbm, wb_hbm, bb_hbm, out_hbm,
             conn_v, w_v, b_v, rows0, rows1, out_v, sem0, sem1):
    wid = lax.axis_index("s") * NC + lax.axis_index("c")

    pltpu.sync_copy(conn_hbm.at[wid], conn_v)
    pltpu.sync_copy(wb_hbm.at[wid], w_v)
    pltpu.sync_copy(bb_hbm.at[wid], b_v)

    def issue(tt, rows_ref, sem):
        oi = tt // B
        b = lax.rem(tt, B)
        pltpu.async_copy(xf_hbm.at[b].at[conn_v.at[oi]], rows_ref, sem)

    def wait_rows(rows_ref, sem):
        pltpu.make_async_copy(xf_hbm.at[0].at[pl.ds(0, K)], rows_ref, sem).wait()

    def compute(tt, rows_ref):
        oi = tt // B
        b = lax.rem(tt, B)
        bv = b_v[oi]

        ws = [w_v[oi, pl.ds(k * L, L)] for k in range(K)]

        def body(j, _):
            s = pl.ds(j * L, L)
            acc = None
            for g in range(0, K, 8):
                d = [jnp.abs(rows_ref[g + k, s] - ws[g + k]) for k in range(8)]
                t0 = jnp.maximum(jnp.maximum(d[0], d[1]), jnp.maximum(d[2], d[3]))
                t1 = jnp.maximum(jnp.maximum(d[4], d[5]), jnp.maximum(d[6], d[7]))
                t = jnp.maximum(t0, t1)
                acc = t if acc is None else jnp.maximum(acc, t)
            out_v[b, oi, s] = acc + bv
            return 0

        lax.fori_loop(0, NJ, body, 0)

    issue(0, rows0, sem0)

    def tbody(i, _):
        t0 = i * 2

        @pl.when(t0 + 1 < T)
        def _():
            issue(t0 + 1, rows1, sem1)

        wait_rows(rows0, sem0)
        compute(t0, rows0)

        @pl.when(t0 + 2 < T)
        def _():
            issue(t0 + 2, rows0, sem0)

        wait_rows(rows1, sem1)
        compute(t0 + 1, rows1)
        return 0

    lax.fori_loop(0, T // 2, tbody, 0)

    pltpu.sync_copy(out_v, out_hbm.at[:, wid])


@jax.jit
def _dist_conv(xf, conn3, w_b, bias_b):
    mesh = plsc.VectorSubcoreMesh(core_axis_name="c", subcore_axis_name="s")
    call = functools.partial(
        pl.kernel,
        out_type=jax.ShapeDtypeStruct((B, NW, OPW, HW), jnp.float32),
        mesh=mesh,
        scratch_types=[
            pltpu.VMEM((OPW, K), jnp.int32),
            pltpu.VMEM((OPW, K * L), jnp.float32),
            pltpu.VMEM((OPW, L), jnp.float32),
            pltpu.VMEM((K, HWP), jnp.float32),
            pltpu.VMEM((K, HWP), jnp.float32),
            pltpu.VMEM((B, OPW, HW), jnp.float32),
            pltpu.SemaphoreType.DMA,
            pltpu.SemaphoreType.DMA,
        ],
    )(_sc_body)
    return call(xf, conn3, w_b, bias_b)


def kernel(x, conn, weights, bias):
    xf = jnp.pad(x.reshape(B, CIN, HW), ((0, 0), (0, 0), (0, HWP - HW)))
    conn3 = conn.reshape(NW, OPW, K)
    w_b = jnp.broadcast_to(weights[:, :, None], (COUT, K, L)).reshape(NW, OPW, K * L)
    bias_b = jnp.broadcast_to(bias.reshape(COUT, 1), (COUT, L)).reshape(NW, OPW, L)
    out = _dist_conv(xf, conn3, w_b, bias_b)
    return out.reshape(B, COUT, H, W)
